# baseline (speedup 1.0000x reference)
import functools

import jax
import jax.numpy as jnp
from jax import lax
from jax.experimental import pallas as pl
from jax.experimental.pallas import tpu as pltpu
from jax.experimental.pallas import tpu_sc as plsc

NW = 32
EC = 4096
LANES = 16


@functools.lru_cache(maxsize=None)
def _make_sc_spmm_t(n, c, ct, groups, nchunk):
    mesh = plsc.VectorSubcoreMesh(core_axis_name="c", subcore_axis_name="s")
    tiles_per_group = c // ct
    active = groups * tiles_per_group
    chunks_per_group = nchunk // groups

    @functools.partial(
        pl.kernel,
        out_type=jax.ShapeDtypeStruct((groups * c, n), jnp.float32),
        mesh=mesh,
        scratch_types=[
            pltpu.VMEM((n,), jnp.float32),
            pltpu.VMEM((n,), jnp.float32),
            pltpu.VMEM((n,), jnp.float32),
            pltpu.VMEM((n,), jnp.float32),
            pltpu.VMEM((2, EC), jnp.int32),
            pltpu.VMEM((2, EC), jnp.int32),
            pltpu.VMEM((2, EC), jnp.float32),
            pltpu.SemaphoreType.DMA,
            pltpu.SemaphoreType.DMA,
            pltpu.SemaphoreType.DMA,
            pltpu.SemaphoreType.DMA,
            pltpu.SemaphoreType.DMA,
            pltpu.SemaphoreType.DMA,
        ],
        compiler_params=pltpu.CompilerParams(
            use_tc_tiling_on_sc=False, needs_layout_passes=False),
    )
    def spmm(tt_hbm, cols_hbm, rows_hbm, w_hbm, out_hbm,
             tc0, tc1, ac0, ac1, cbuf, rbuf, wbuf,
             cs0, cs1, rs0, rs1, ws0, ws1):
        cid = lax.axis_index("c")
        sid = lax.axis_index("s")
        wid = sid * 2 + cid
        tcs = (tc0, tc1)[:ct]
        acs = (ac0, ac1)[:ct]
        csem = (cs0, cs1)
        rsem = (rs0, rs1)
        wsem = (ws0, ws1)

        @pl.when(wid < active)
        def _body():
            g = wid // tiles_per_group
            slot = wid % tiles_per_group
            base = g * chunks_per_group

            for k in range(ct):
                pltpu.sync_copy(tt_hbm.at[slot * ct + k], tcs[k])

            zero = jnp.zeros((LANES,), jnp.float32)

            def zbody(i, carry):
                for k in range(ct):
                    acs[k][pl.ds(i * LANES, LANES)] = zero
                return carry

            lax.fori_loop(0, n // LANES, zbody, 0)

            def stage(j, b):
                off = (base + j) * EC
                pltpu.async_copy(cols_hbm.at[pl.ds(off, EC)], cbuf.at[b],
                                 csem[b])
                pltpu.async_copy(rows_hbm.at[pl.ds(off, EC)], rbuf.at[b],
                                 rsem[b])
                pltpu.async_copy(w_hbm.at[pl.ds(off, EC)], wbuf.at[b],
                                 wsem[b])

            def swait(j, b):
                off = (base + j) * EC
                pltpu.make_async_copy(cols_hbm.at[pl.ds(off, EC)],
                                      cbuf.at[b], csem[b]).wait()
                pltpu.make_async_copy(rows_hbm.at[pl.ds(off, EC)],
                                      rbuf.at[b], rsem[b]).wait()
                pltpu.make_async_copy(w_hbm.at[pl.ds(off, EC)],
                                      wbuf.at[b], wsem[b]).wait()

            def inner(b):
                unroll = 4

                def vbody(u, carry):
                    colvs, rowvs, wvs = [], [], []
                    for q in range(unroll):
                        o = (u * unroll + q) * LANES
                        colvs.append(cbuf[b, pl.ds(o, LANES)])
                        rowvs.append(rbuf[b, pl.ds(o, LANES)])
                        wvs.append(wbuf[b, pl.ds(o, LANES)])
                    for k in range(ct):
                        gs = [plsc.load_gather(tcs[k], [colvs[q]])
                              for q in range(unroll)]
                        for q in range(unroll):
                            plsc.addupdate_scatter(
                                acs[k], [rowvs[q]], gs[q] * wvs[q])
                    return carry

                lax.fori_loop(0, EC // LANES // unroll, vbody, 0)

            stage(0, 0)

            def pair_body(jh, carry):
                j0 = 2 * jh
                j1 = j0 + 1
                stage(j1, 1)
                swait(j0, 0)
                inner(0)
                stage(lax.min(j0 + 2, chunks_per_group - 1), 0)
                swait(j1, 1)
                inner(1)
                return carry

            lax.fori_loop(0, chunks_per_group // 2, pair_body, 0)
            swait(chunks_per_group - 1, 0)

            for k in range(ct):
                pltpu.sync_copy(acs[k], out_hbm.at[g * c + slot * ct + k])

    return spmm


def _prep_body(xt_ref, w0t_ref, wadjt_ref, out_ref):
    h = w0t_ref.shape[0]
    out_ref[:h, :] = jnp.dot(w0t_ref[...], xt_ref[...],
                             preferred_element_type=jnp.float32)
    out_ref[h:, :] = wadjt_ref[...]


def _mid1_body(s0t_ref, w1t_ref, out_ref, adjnt_ref):
    h = s0t_ref.shape[0] // 2
    h1t = jax.nn.relu(s0t_ref[:h, :]) + jax.nn.relu(s0t_ref[h:, :])
    out_ref[...] = jnp.dot(w1t_ref[...], h1t,
                           preferred_element_type=jnp.float32)
    adjnt_ref[...] = s0t_ref[h:, :].T


def _adjrec_body(adjnt_ref, wrec_ref, out_ref):
    hadj = jax.nn.relu(adjnt_ref[...])
    out_ref[...] = jax.nn.relu(
        jnp.dot(hadj, wrec_ref[...], preferred_element_type=jnp.float32))


def _mid2_body(s1t_ref, s0t_ref, w2t_ref, out_ref):
    h = s0t_ref.shape[0] // 2
    hadj_t = jax.nn.relu(s0t_ref[h:, :])
    h2t = jax.nn.relu(s1t_ref[...]) + hadj_t
    out_ref[...] = jnp.dot(w2t_ref[...], h2t,
                           preferred_element_type=jnp.float32)


def _tr_body(p2t_ref, out_ref):
    out_ref[...] = p2t_ref[...].T


def _zzt_body(p2nt_ref, p2nt_full_ref, out_ref):
    d = p2nt_ref.shape[1] // 2
    zr = p2nt_ref[:, :d] + p2nt_ref[:, d:]
    znt = p2nt_full_ref[:, :d] + p2nt_full_ref[:, d:]
    out_ref[...] = lax.dot_general(
        zr, znt, (((1,), (1,)), ((), ())),
        preferred_element_type=jnp.float32)


def _tc_prep(xt, W0t, W_adjt):
    h, n = W_adjt.shape
    return pl.pallas_call(
        _prep_body,
        out_shape=jax.ShapeDtypeStruct((2 * h, n), jnp.float32),
    )(xt, W0t, W_adjt)


def _tc_mid1(s0t, W1t):
    n = s0t.shape[1]
    h = W1t.shape[0]
    return pl.pallas_call(
        _mid1_body,
        out_shape=[jax.ShapeDtypeStruct((h, n), jnp.float32),
                   jax.ShapeDtypeStruct((n, h), jnp.float32)],
    )(s0t, W1t)


def _tc_adjrec(adjnt, W_rec):
    n, h = adjnt.shape
    br = 400
    gr = n // br
    return pl.pallas_call(
        _adjrec_body,
        grid=(gr,),
        in_specs=[
            pl.BlockSpec((br, h), lambda i: (i, 0)),
            pl.BlockSpec((h, n), lambda i: (0, 0)),
        ],
        out_specs=pl.BlockSpec((br, n), lambda i: (i, 0)),
        out_shape=jax.ShapeDtypeStruct((n, n), jnp.float32),
    )(adjnt, W_rec)


def _tc_mid2(s1t, s0t, W2t):
    n = s0t.shape[1]
    d = W2t.shape[0]
    return pl.pallas_call(
        _mid2_body,
        out_shape=jax.ShapeDtypeStruct((d, n), jnp.float32),
    )(s1t, s0t, W2t)


def _tc_tr(p2t):
    c, n = p2t.shape
    return pl.pallas_call(
        _tr_body,
        out_shape=jax.ShapeDtypeStruct((n, c), jnp.float32),
    )(p2t)


def _tc_zzt(p2nt):
    n, c2 = p2nt.shape
    br = 400
    gr = n // br
    return pl.pallas_call(
        _zzt_body,
        grid=(gr,),
        in_specs=[
            pl.BlockSpec((br, c2), lambda i: (i, 0)),
            pl.BlockSpec((n, c2), lambda i: (0, 0)),
        ],
        out_specs=pl.BlockSpec((br, n), lambda i: (i, 0)),
        out_shape=jax.ShapeDtypeStruct((n, n), jnp.float32),
    )(p2nt, p2nt)


def _edge_split(edge_index, edge_weight):
    e = edge_index.shape[1]
    nchunk = 4 * (-(-e // (4 * EC)))
    epad = nchunk * EC - e
    rows1 = jnp.pad(edge_index[0], (0, epad))
    cols1 = jnp.pad(edge_index[1], (0, epad))
    w1 = jnp.pad(edge_weight, (0, epad))
    return cols1, rows1, w1, nchunk


def kernel(x, edge_index, edge_weight, W0, W_adj, W_rec, W1, W2):
    n = x.shape[0]
    h = W0.shape[1]
    d = W2.shape[1]

    cols1, rows1, w1, nchunk = _edge_split(edge_index, edge_weight)

    xt = x.T
    W0t = W0.T
    W_adjt = W_adj.T
    W1t = W1.T
    W2t = W2.T

    tcat_t = _tc_prep(xt, W0t, W_adjt)
    s0t = _make_sc_spmm_t(n, 2 * h, 2, 1, nchunk)(
        tcat_t, cols1, rows1, w1)
    hw1t, adjnt = _tc_mid1(s0t, W1t)
    s1t = _make_sc_spmm_t(n, h, 1, 1, nchunk)(
        hw1t, cols1, rows1, w1)
    adj_rec = _tc_adjrec(adjnt, W_rec)
    hw2t = _tc_mid2(s1t, s0t, W2t)
    p2t = _make_sc_spmm_t(n, d, 1, 2, nchunk)(
        hw2t, cols1, rows1, w1)
    p2nt = _tc_tr(p2t)
    zzt = _tc_zzt(p2nt)

    return (jnp.reshape(zzt, (-1,)), adj_rec)


# speedup vs baseline: 1.0069x; 1.0069x over previous
import functools

import jax
import jax.numpy as jnp
from jax import lax
from jax.experimental import pallas as pl
from jax.experimental.pallas import tpu as pltpu
from jax.experimental.pallas import tpu_sc as plsc

NW = 32
EC = 4096
LANES = 16


@functools.lru_cache(maxsize=None)
def _make_sc_spmm_t(n, c, ct, groups, nchunk):
    mesh = plsc.VectorSubcoreMesh(core_axis_name="c", subcore_axis_name="s")
    tiles_per_group = c // ct
    active = groups * tiles_per_group
    chunks_per_group = nchunk // groups

    @functools.partial(
        pl.kernel,
        out_type=jax.ShapeDtypeStruct((groups * c, n), jnp.float32),
        mesh=mesh,
        scratch_types=[
            pltpu.VMEM((n,), jnp.float32),
            pltpu.VMEM((n,), jnp.float32),
            pltpu.VMEM((n,), jnp.float32),
            pltpu.VMEM((n,), jnp.float32),
            pltpu.VMEM((2, EC), jnp.int32),
            pltpu.VMEM((2, EC), jnp.int32),
            pltpu.VMEM((2, EC), jnp.float32),
            pltpu.SemaphoreType.DMA,
            pltpu.SemaphoreType.DMA,
            pltpu.SemaphoreType.DMA,
            pltpu.SemaphoreType.DMA,
            pltpu.SemaphoreType.DMA,
            pltpu.SemaphoreType.DMA,
        ],
        compiler_params=pltpu.CompilerParams(
            use_tc_tiling_on_sc=False, needs_layout_passes=False),
    )
    def spmm(tt_hbm, cols_hbm, rows_hbm, w_hbm, out_hbm,
             tc0, tc1, ac0, ac1, cbuf, rbuf, wbuf,
             cs0, cs1, rs0, rs1, ws0, ws1):
        cid = lax.axis_index("c")
        sid = lax.axis_index("s")
        wid = sid * 2 + cid
        tcs = (tc0, tc1)[:ct]
        acs = (ac0, ac1)[:ct]
        csem = (cs0, cs1)
        rsem = (rs0, rs1)
        wsem = (ws0, ws1)

        @pl.when(wid < active)
        def _body():
            g = wid // tiles_per_group
            slot = wid % tiles_per_group
            base = g * chunks_per_group

            for k in range(ct):
                pltpu.sync_copy(tt_hbm.at[slot * ct + k], tcs[k])

            zero = jnp.zeros((LANES,), jnp.float32)

            def zbody(i, carry):
                for k in range(ct):
                    acs[k][pl.ds(i * LANES, LANES)] = zero
                return carry

            lax.fori_loop(0, n // LANES, zbody, 0)

            def stage(j, b):
                off = (base + j) * EC
                pltpu.async_copy(cols_hbm.at[pl.ds(off, EC)], cbuf.at[b],
                                 csem[b])
                pltpu.async_copy(rows_hbm.at[pl.ds(off, EC)], rbuf.at[b],
                                 rsem[b])
                pltpu.async_copy(w_hbm.at[pl.ds(off, EC)], wbuf.at[b],
                                 wsem[b])

            def swait(j, b):
                off = (base + j) * EC
                pltpu.make_async_copy(cols_hbm.at[pl.ds(off, EC)],
                                      cbuf.at[b], csem[b]).wait()
                pltpu.make_async_copy(rows_hbm.at[pl.ds(off, EC)],
                                      rbuf.at[b], rsem[b]).wait()
                pltpu.make_async_copy(w_hbm.at[pl.ds(off, EC)],
                                      wbuf.at[b], wsem[b]).wait()

            def inner(b):
                unroll = 8 if ct == 1 else 4

                def vbody(u, carry):
                    colvs, rowvs, wvs = [], [], []
                    for q in range(unroll):
                        o = (u * unroll + q) * LANES
                        colvs.append(cbuf[b, pl.ds(o, LANES)])
                        rowvs.append(rbuf[b, pl.ds(o, LANES)])
                        wvs.append(wbuf[b, pl.ds(o, LANES)])
                    for k in range(ct):
                        gs = [plsc.load_gather(tcs[k], [colvs[q]])
                              for q in range(unroll)]
                        for q in range(unroll):
                            plsc.addupdate_scatter(
                                acs[k], [rowvs[q]], gs[q] * wvs[q])
                    return carry

                lax.fori_loop(0, EC // LANES // unroll, vbody, 0)

            stage(0, 0)

            def pair_body(jh, carry):
                j0 = 2 * jh
                j1 = j0 + 1
                stage(j1, 1)
                swait(j0, 0)
                inner(0)
                stage(lax.min(j0 + 2, chunks_per_group - 1), 0)
                swait(j1, 1)
                inner(1)
                return carry

            lax.fori_loop(0, chunks_per_group // 2, pair_body, 0)
            swait(chunks_per_group - 1, 0)

            for k in range(ct):
                pltpu.sync_copy(acs[k], out_hbm.at[g * c + slot * ct + k])

    return spmm


def _prep_body(xt_ref, w0t_ref, wadjt_ref, out_ref):
    h = w0t_ref.shape[0]
    out_ref[:h, :] = jnp.dot(w0t_ref[...], xt_ref[...],
                             preferred_element_type=jnp.float32)
    out_ref[h:, :] = wadjt_ref[...]


def _mid1_body(s0t_ref, w1t_ref, out_ref, adjnt_ref):
    h = s0t_ref.shape[0] // 2
    h1t = jax.nn.relu(s0t_ref[:h, :]) + jax.nn.relu(s0t_ref[h:, :])
    out_ref[...] = jnp.dot(w1t_ref[...], h1t,
                           preferred_element_type=jnp.float32)
    adjnt_ref[...] = s0t_ref[h:, :].T


def _adjrec_body(adjnt_ref, wrec_ref, out_ref):
    hadj = jax.nn.relu(adjnt_ref[...])
    out_ref[...] = jax.nn.relu(
        jnp.dot(hadj, wrec_ref[...], preferred_element_type=jnp.float32))


def _mid2_body(s1t_ref, s0t_ref, w2t_ref, out_ref):
    h = s0t_ref.shape[0] // 2
    hadj_t = jax.nn.relu(s0t_ref[h:, :])
    h2t = jax.nn.relu(s1t_ref[...]) + hadj_t
    out_ref[...] = jnp.dot(w2t_ref[...], h2t,
                           preferred_element_type=jnp.float32)


def _tr_body(p2t_ref, out_ref):
    out_ref[...] = p2t_ref[...].T


def _zzt_body(p2nt_ref, p2nt_full_ref, out_ref):
    d = p2nt_ref.shape[1] // 2
    zr = p2nt_ref[:, :d] + p2nt_ref[:, d:]
    znt = p2nt_full_ref[:, :d] + p2nt_full_ref[:, d:]
    out_ref[...] = lax.dot_general(
        zr, znt, (((1,), (1,)), ((), ())),
        preferred_element_type=jnp.float32)


def _tc_prep(xt, W0t, W_adjt):
    h, n = W_adjt.shape
    return pl.pallas_call(
        _prep_body,
        out_shape=jax.ShapeDtypeStruct((2 * h, n), jnp.float32),
    )(xt, W0t, W_adjt)


def _tc_mid1(s0t, W1t):
    n = s0t.shape[1]
    h = W1t.shape[0]
    return pl.pallas_call(
        _mid1_body,
        out_shape=[jax.ShapeDtypeStruct((h, n), jnp.float32),
                   jax.ShapeDtypeStruct((n, h), jnp.float32)],
    )(s0t, W1t)


def _tc_adjrec(adjnt, W_rec):
    n, h = adjnt.shape
    br = 400
    gr = n // br
    return pl.pallas_call(
        _adjrec_body,
        grid=(gr,),
        in_specs=[
            pl.BlockSpec((br, h), lambda i: (i, 0)),
            pl.BlockSpec((h, n), lambda i: (0, 0)),
        ],
        out_specs=pl.BlockSpec((br, n), lambda i: (i, 0)),
        out_shape=jax.ShapeDtypeStruct((n, n), jnp.float32),
    )(adjnt, W_rec)


def _tc_mid2(s1t, s0t, W2t):
    n = s0t.shape[1]
    d = W2t.shape[0]
    return pl.pallas_call(
        _mid2_body,
        out_shape=jax.ShapeDtypeStruct((d, n), jnp.float32),
    )(s1t, s0t, W2t)


def _tc_tr(p2t):
    c, n = p2t.shape
    return pl.pallas_call(
        _tr_body,
        out_shape=jax.ShapeDtypeStruct((n, c), jnp.float32),
    )(p2t)


def _tc_zzt(p2nt):
    n, c2 = p2nt.shape
    br = 400
    gr = n // br
    return pl.pallas_call(
        _zzt_body,
        grid=(gr,),
        in_specs=[
            pl.BlockSpec((br, c2), lambda i: (i, 0)),
            pl.BlockSpec((n, c2), lambda i: (0, 0)),
        ],
        out_specs=pl.BlockSpec((br, n), lambda i: (i, 0)),
        out_shape=jax.ShapeDtypeStruct((n, n), jnp.float32),
    )(p2nt, p2nt)


def _edge_split(edge_index, edge_weight):
    e = edge_index.shape[1]
    nchunk = 4 * (-(-e // (4 * EC)))
    epad = nchunk * EC - e
    rows1 = jnp.pad(edge_index[0], (0, epad))
    cols1 = jnp.pad(edge_index[1], (0, epad))
    w1 = jnp.pad(edge_weight, (0, epad))
    return cols1, rows1, w1, nchunk


def kernel(x, edge_index, edge_weight, W0, W_adj, W_rec, W1, W2):
    n = x.shape[0]
    h = W0.shape[1]
    d = W2.shape[1]

    cols1, rows1, w1, nchunk = _edge_split(edge_index, edge_weight)

    xt = x.T
    W0t = W0.T
    W_adjt = W_adj.T
    W1t = W1.T
    W2t = W2.T

    tcat_t = _tc_prep(xt, W0t, W_adjt)
    s0t = _make_sc_spmm_t(n, 2 * h, 2, 1, nchunk)(
        tcat_t, cols1, rows1, w1)
    hw1t, adjnt = _tc_mid1(s0t, W1t)
    s1t = _make_sc_spmm_t(n, h, 1, 1, nchunk)(
        hw1t, cols1, rows1, w1)
    adj_rec = _tc_adjrec(adjnt, W_rec)
    hw2t = _tc_mid2(s1t, s0t, W2t)
    p2t = _make_sc_spmm_t(n, d, 1, 2, nchunk)(
        hw2t, cols1, rows1, w1)
    p2nt = _tc_tr(p2t)
    zzt = _tc_zzt(p2nt)

    return (jnp.reshape(zzt, (-1,)), adj_rec)


# speedup vs baseline: 1.0343x; 1.0273x over previous
import functools

import jax
import jax.numpy as jnp
from jax import lax
from jax.experimental import pallas as pl
from jax.experimental.pallas import tpu as pltpu
from jax.experimental.pallas import tpu_sc as plsc

NW = 32
EC = 4096
LANES = 16


@functools.lru_cache(maxsize=None)
def _make_sc_spmm_t(n, c, ct, groups, nchunk):
    mesh = plsc.VectorSubcoreMesh(core_axis_name="c", subcore_axis_name="s")
    tiles_per_group = c // ct
    active = groups * tiles_per_group
    chunks_per_group = nchunk // groups

    @functools.partial(
        pl.kernel,
        out_type=jax.ShapeDtypeStruct((groups * c, n), jnp.float32),
        mesh=mesh,
        scratch_types=[
            pltpu.VMEM((n,), jnp.float32),
            pltpu.VMEM((n,), jnp.float32),
            pltpu.VMEM((n,), jnp.float32),
            pltpu.VMEM((n,), jnp.float32),
            pltpu.VMEM((2, EC), jnp.int32),
            pltpu.VMEM((2, EC), jnp.int32),
            pltpu.VMEM((2, EC), jnp.float32),
            pltpu.SemaphoreType.DMA,
            pltpu.SemaphoreType.DMA,
            pltpu.SemaphoreType.DMA,
            pltpu.SemaphoreType.DMA,
            pltpu.SemaphoreType.DMA,
            pltpu.SemaphoreType.DMA,
        ],
        compiler_params=pltpu.CompilerParams(
            use_tc_tiling_on_sc=False, needs_layout_passes=False),
    )
    def spmm(tt_hbm, cols_hbm, rows_hbm, w_hbm, out_hbm,
             tc0, tc1, ac0, ac1, cbuf, rbuf, wbuf,
             cs0, cs1, rs0, rs1, ws0, ws1):
        cid = lax.axis_index("c")
        sid = lax.axis_index("s")
        wid = sid * 2 + cid
        tcs = (tc0, tc1)[:ct]
        acs = (ac0, ac1)[:ct]
        csem = (cs0, cs1)
        rsem = (rs0, rs1)
        wsem = (ws0, ws1)
        packed = ct == 1
        sh = (n - 1).bit_length()
        mask = (1 << sh) - 1

        @pl.when(wid < active)
        def _body():
            g = wid // tiles_per_group
            slot = wid % tiles_per_group
            base = g * chunks_per_group

            for k in range(ct):
                pltpu.sync_copy(tt_hbm.at[slot * ct + k], tcs[k])

            zero = jnp.zeros((LANES,), jnp.float32)

            def zbody(i, carry):
                for k in range(ct):
                    acs[k][pl.ds(i * LANES, LANES)] = zero
                return carry

            lax.fori_loop(0, n // LANES, zbody, 0)

            def stage(j, b):
                off = (base + j) * EC
                pltpu.async_copy(cols_hbm.at[pl.ds(off, EC)], cbuf.at[b],
                                 csem[b])
                if not packed:
                    pltpu.async_copy(rows_hbm.at[pl.ds(off, EC)], rbuf.at[b],
                                     rsem[b])
                pltpu.async_copy(w_hbm.at[pl.ds(off, EC)], wbuf.at[b],
                                 wsem[b])

            def swait(j, b):
                off = (base + j) * EC
                pltpu.make_async_copy(cols_hbm.at[pl.ds(off, EC)],
                                      cbuf.at[b], csem[b]).wait()
                if not packed:
                    pltpu.make_async_copy(rows_hbm.at[pl.ds(off, EC)],
                                          rbuf.at[b], rsem[b]).wait()
                pltpu.make_async_copy(w_hbm.at[pl.ds(off, EC)],
                                      wbuf.at[b], wsem[b]).wait()

            def inner(b):
                unroll = 8 if ct == 1 else 4

                def vbody(u, carry):
                    colvs, rowvs, wvs = [], [], []
                    for q in range(unroll):
                        o = (u * unroll + q) * LANES
                        if packed:
                            pv = cbuf[b, pl.ds(o, LANES)]
                            colvs.append(pv & mask)
                            rowvs.append(lax.shift_right_logical(pv, sh))
                        else:
                            colvs.append(cbuf[b, pl.ds(o, LANES)])
                            rowvs.append(rbuf[b, pl.ds(o, LANES)])
                        wvs.append(wbuf[b, pl.ds(o, LANES)])
                    for k in range(ct):
                        gs = [plsc.load_gather(tcs[k], [colvs[q]])
                              for q in range(unroll)]
                        for q in range(unroll):
                            plsc.addupdate_scatter(
                                acs[k], [rowvs[q]], gs[q] * wvs[q])
                    return carry

                lax.fori_loop(0, EC // LANES // unroll, vbody, 0)

            stage(0, 0)

            def pair_body(jh, carry):
                j0 = 2 * jh
                j1 = j0 + 1
                stage(j1, 1)
                swait(j0, 0)
                inner(0)
                stage(lax.min(j0 + 2, chunks_per_group - 1), 0)
                swait(j1, 1)
                inner(1)
                return carry

            lax.fori_loop(0, chunks_per_group // 2, pair_body, 0)
            swait(chunks_per_group - 1, 0)

            for k in range(ct):
                pltpu.sync_copy(acs[k], out_hbm.at[g * c + slot * ct + k])

    return spmm


def _prep_body(xt_ref, w0t_ref, wadjt_ref, out_ref):
    h = w0t_ref.shape[0]
    out_ref[:h, :] = jnp.dot(w0t_ref[...], xt_ref[...],
                             preferred_element_type=jnp.float32)
    out_ref[h:, :] = wadjt_ref[...]


def _mid1_body(s0t_ref, w1t_ref, out_ref, adjnt_ref):
    h = s0t_ref.shape[0] // 2
    h1t = jax.nn.relu(s0t_ref[:h, :]) + jax.nn.relu(s0t_ref[h:, :])
    out_ref[...] = jnp.dot(w1t_ref[...], h1t,
                           preferred_element_type=jnp.float32)
    adjnt_ref[...] = s0t_ref[h:, :].T


def _adjrec_body(adjnt_ref, wrec_ref, out_ref):
    hadj = jax.nn.relu(adjnt_ref[...])
    out_ref[...] = jax.nn.relu(
        jnp.dot(hadj, wrec_ref[...], preferred_element_type=jnp.float32))


def _mid2_body(s1t_ref, s0t_ref, w2t_ref, out_ref):
    h = s0t_ref.shape[0] // 2
    hadj_t = jax.nn.relu(s0t_ref[h:, :])
    h2t = jax.nn.relu(s1t_ref[...]) + hadj_t
    out_ref[...] = jnp.dot(w2t_ref[...], h2t,
                           preferred_element_type=jnp.float32)


def _tr_body(p2t_ref, out_ref):
    out_ref[...] = p2t_ref[...].T


def _zzt_body(p2nt_ref, p2nt_full_ref, out_ref):
    d = p2nt_ref.shape[1] // 2
    zr = p2nt_ref[:, :d] + p2nt_ref[:, d:]
    znt = p2nt_full_ref[:, :d] + p2nt_full_ref[:, d:]
    out_ref[...] = lax.dot_general(
        zr, znt, (((1,), (1,)), ((), ())),
        preferred_element_type=jnp.float32)


def _tc_prep(xt, W0t, W_adjt):
    h, n = W_adjt.shape
    return pl.pallas_call(
        _prep_body,
        out_shape=jax.ShapeDtypeStruct((2 * h, n), jnp.float32),
    )(xt, W0t, W_adjt)


def _tc_mid1(s0t, W1t):
    n = s0t.shape[1]
    h = W1t.shape[0]
    return pl.pallas_call(
        _mid1_body,
        out_shape=[jax.ShapeDtypeStruct((h, n), jnp.float32),
                   jax.ShapeDtypeStruct((n, h), jnp.float32)],
    )(s0t, W1t)


def _tc_adjrec(adjnt, W_rec):
    n, h = adjnt.shape
    br = 400
    gr = n // br
    return pl.pallas_call(
        _adjrec_body,
        grid=(gr,),
        in_specs=[
            pl.BlockSpec((br, h), lambda i: (i, 0)),
            pl.BlockSpec((h, n), lambda i: (0, 0)),
        ],
        out_specs=pl.BlockSpec((br, n), lambda i: (i, 0)),
        out_shape=jax.ShapeDtypeStruct((n, n), jnp.float32),
    )(adjnt, W_rec)


def _tc_mid2(s1t, s0t, W2t):
    n = s0t.shape[1]
    d = W2t.shape[0]
    return pl.pallas_call(
        _mid2_body,
        out_shape=jax.ShapeDtypeStruct((d, n), jnp.float32),
    )(s1t, s0t, W2t)


def _tc_tr(p2t):
    c, n = p2t.shape
    return pl.pallas_call(
        _tr_body,
        out_shape=jax.ShapeDtypeStruct((n, c), jnp.float32),
    )(p2t)


def _tc_zzt(p2nt):
    n, c2 = p2nt.shape
    br = 400
    gr = n // br
    return pl.pallas_call(
        _zzt_body,
        grid=(gr,),
        in_specs=[
            pl.BlockSpec((br, c2), lambda i: (i, 0)),
            pl.BlockSpec((n, c2), lambda i: (0, 0)),
        ],
        out_specs=pl.BlockSpec((br, n), lambda i: (i, 0)),
        out_shape=jax.ShapeDtypeStruct((n, n), jnp.float32),
    )(p2nt, p2nt)


def _edge_split(edge_index, edge_weight):
    e = edge_index.shape[1]
    nchunk = 4 * (-(-e // (4 * EC)))
    epad = nchunk * EC - e
    rows1 = jnp.pad(edge_index[0], (0, epad))
    cols1 = jnp.pad(edge_index[1], (0, epad))
    w1 = jnp.pad(edge_weight, (0, epad))
    return cols1, rows1, w1, nchunk


def kernel(x, edge_index, edge_weight, W0, W_adj, W_rec, W1, W2):
    n = x.shape[0]
    h = W0.shape[1]
    d = W2.shape[1]

    cols1, rows1, w1, nchunk = _edge_split(edge_index, edge_weight)
    sh = (n - 1).bit_length()
    pk1 = cols1 | (rows1 << sh)

    xt = x.T
    W0t = W0.T
    W_adjt = W_adj.T
    W1t = W1.T
    W2t = W2.T

    tcat_t = _tc_prep(xt, W0t, W_adjt)
    s0t = _make_sc_spmm_t(n, 2 * h, 2, 1, nchunk)(
        tcat_t, cols1, rows1, w1)
    hw1t, adjnt = _tc_mid1(s0t, W1t)
    s1t = _make_sc_spmm_t(n, h, 1, 1, nchunk)(
        hw1t, pk1, rows1, w1)
    adj_rec = _tc_adjrec(adjnt, W_rec)
    hw2t = _tc_mid2(s1t, s0t, W2t)
    p2t = _make_sc_spmm_t(n, d, 1, 2, nchunk)(
        hw2t, pk1, rows1, w1)
    p2nt = _tc_tr(p2t)
    zzt = _tc_zzt(p2nt)

    return (jnp.reshape(zzt, (-1,)), adj_rec)
